# baseline (speedup 1.0000x reference)
import functools

import jax
import jax.numpy as jnp
from jax import lax
from jax.experimental import pallas as pl
from jax.experimental.pallas import tpu as pltpu
from jax.experimental.pallas import tpu_sc as plsc

_N_E = 512
_E_DIM = 256
_BETA = 0.25
_TOKENS = 16384
_SEG = _TOKENS // 2
_TB = 2048
_GRID = _SEG // _TB

_NC = 2
_NS = 16
_NW = _NC * _NS
_BPW = _SEG // _NW
_CH = 128
_NCH = _BPW // _CH
_NBUF = 2


def _dist_kernel(with_zq, z_ref, e_ref, cin_ref, min_ref, idx_ref,
                 counts_ref, msum_ref, loss_ref, perp_ref, zq_ref):
    i = pl.program_id(0)
    z = z_ref[...]
    e = e_ref[...]
    zsq = jnp.sum(z * z, axis=1, keepdims=True)
    esq = jnp.sum(e * e, axis=1)
    mm = lax.dot_general(z, e, (((1,), (1,)), ((), ())),
                         preferred_element_type=jnp.float32)
    d = (zsq + esq[None, :]) - 2.0 * mm
    dmin = jnp.min(d, axis=1, keepdims=True)
    iota_f = lax.broadcasted_iota(jnp.int32, (_TB, _N_E), 1).astype(jnp.float32)
    idxf = jnp.min(jnp.where(d == dmin, iota_f, float(_N_E)),
                   axis=1, keepdims=True)
    idx_ref[...] = idxf.astype(jnp.int32).reshape(_TB // 128, 128)

    onehot = (idxf == iota_f).astype(jnp.float32)
    cpart = jnp.sum(onehot, axis=0)
    if with_zq:
        zq_ref[...] = lax.dot_general(onehot, e, (((1,), (0,)), ((), ())),
                                      preferred_element_type=jnp.float32)

    @pl.when(i == 0)
    def _init():
        counts_ref[...] = jnp.zeros_like(counts_ref)
        msum_ref[...] = jnp.zeros_like(msum_ref)

    counts_ref[...] += cpart[None, :]
    msum_ref[...] += jnp.reshape(jnp.sum(dmin), (1, 1))

    @pl.when(i == _GRID - 1)
    def _epilogue():
        if with_zq:
            g = lax.dot_general(e, e, (((1,), (1,)), ((), ())),
                                preferred_element_type=jnp.float32)
            sq = esq[:, None] + esq[None, :] - 2.0 * g
            ed = jnp.sqrt(jnp.maximum(sq, 0.0))
            ri = lax.broadcasted_iota(jnp.int32, (_N_E, _N_E), 0)
            ci = lax.broadcasted_iota(jnp.int32, (_N_E, _N_E), 1)
            tril = jnp.where(ri >= ci, ed, 0.0)
            e_loss = jnp.exp(-(jnp.sum(tril) / float(_N_E * _N_E)) / 0.1)

            emean = ((counts_ref[0, :] + cin_ref[0, :])
                     * (1.0 / float(_TOKENS)))
            perp = jnp.exp(-jnp.sum(emean * jnp.log(emean + 1e-10)))

            mse = ((msum_ref[0, 0] + min_ref[0, 0])
                   / float(_TOKENS * _E_DIM))
            loss_ref[...] = jnp.reshape((1.0 + _BETA) * mse + e_loss, (1, 1))
            perp_ref[...] = jnp.reshape(perp, (1, 1))
        else:
            loss_ref[...] = jnp.zeros_like(loss_ref)
            perp_ref[...] = jnp.zeros_like(perp_ref)


def _gather_body(table_hbm, idx_hbm, out_hbm, idx_v, rows_a, rows_b,
                 gsem_a, gsem_b, osem_a, osem_b):
    wid = lax.axis_index("s") * _NC + lax.axis_index("c")
    base = wid * _BPW
    pltpu.sync_copy(idx_hbm.at[pl.ds(base, _BPW)], idx_v)
    rows = (rows_a, rows_b)
    gsems = (gsem_a, gsem_b)
    osems = (osem_a, osem_b)
    gathers = [None] * _NCH
    scatters = [None] * _NCH
    for c in range(min(_NBUF, _NCH)):
        gathers[c] = pltpu.async_copy(
            table_hbm.at[idx_v.at[pl.ds(c * _CH, _CH)]], rows[c % _NBUF],
            gsems[c % _NBUF])
    for c in range(_NCH):
        b = c % _NBUF
        gathers[c].wait()
        scatters[c] = pltpu.async_copy(
            rows[b], out_hbm.at[pl.ds(base + c * _CH, _CH)], osems[b])
        if c + _NBUF < _NCH:
            scatters[c].wait()
            gathers[c + _NBUF] = pltpu.async_copy(
                table_hbm.at[idx_v.at[pl.ds((c + _NBUF) * _CH, _CH)]],
                rows[b], gsems[b])
    for c in range(max(0, _NCH - _NBUF), _NCH):
        scatters[c].wait()


def _sc_gather(table, idx_flat):
    mesh = plsc.VectorSubcoreMesh(core_axis_name="c", subcore_axis_name="s")
    return pl.kernel(
        _gather_body,
        out_type=jax.ShapeDtypeStruct((_SEG, _E_DIM), jnp.float32),
        mesh=mesh,
        scratch_types=[
            pltpu.VMEM((_BPW,), jnp.int32),
            pltpu.VMEM((_CH, _E_DIM), jnp.float32),
            pltpu.VMEM((_CH, _E_DIM), jnp.float32),
            pltpu.SemaphoreType.DMA,
            pltpu.SemaphoreType.DMA,
            pltpu.SemaphoreType.DMA,
            pltpu.SemaphoreType.DMA,
        ],
    )(table, idx_flat)


def _distance_call(with_zq, seg_off, z_flat, e, cin, min_in):
    zq_shape = (_SEG, _E_DIM) if with_zq else (8, 128)
    return pl.pallas_call(
        functools.partial(_dist_kernel, with_zq),
        grid=(_GRID,),
        in_specs=[
            pl.BlockSpec((_TB, _E_DIM), lambda i: (i + seg_off, 0)),
            pl.BlockSpec((_N_E, _E_DIM), lambda i: (0, 0)),
            pl.BlockSpec((1, _N_E), lambda i: (0, 0)),
            pl.BlockSpec((1, 1), lambda i: (0, 0)),
        ],
        out_specs=[
            pl.BlockSpec((_TB // 128, 128), lambda i: (i, 0)),
            pl.BlockSpec((1, _N_E), lambda i: (0, 0)),
            pl.BlockSpec((1, 1), lambda i: (0, 0)),
            pl.BlockSpec((1, 1), lambda i: (0, 0)),
            pl.BlockSpec((1, 1), lambda i: (0, 0)),
            (pl.BlockSpec((_TB, _E_DIM), lambda i: (i, 0)) if with_zq
             else pl.BlockSpec((8, 128), lambda i: (0, 0))),
        ],
        out_shape=[
            jax.ShapeDtypeStruct((_SEG // 128, 128), jnp.int32),
            jax.ShapeDtypeStruct((1, _N_E), jnp.float32),
            jax.ShapeDtypeStruct((1, 1), jnp.float32),
            jax.ShapeDtypeStruct((1, 1), jnp.float32),
            jax.ShapeDtypeStruct((1, 1), jnp.float32),
            jax.ShapeDtypeStruct(zq_shape, jnp.float32),
        ],
    )(z_flat, e, cin, min_in)


def kernel(z, embedding_weight):
    z_flat = z.reshape(_TOKENS, _E_DIM)
    e = embedding_weight
    zero_c = jnp.zeros((1, _N_E), jnp.float32)
    zero_m = jnp.zeros((1, 1), jnp.float32)

    idx1, counts1, msum1, _, _, _ = _distance_call(
        False, 0, z_flat, e, zero_c, zero_m)
    zq1 = _sc_gather(e, idx1.reshape(_SEG))
    idx2, _, _, loss11, perp11, zq2 = _distance_call(
        True, _GRID, z_flat, e, counts1, msum1)

    zq = jnp.concatenate([zq1, zq2], axis=0)
    idx_flat = jnp.concatenate([idx1, idx2], axis=0).reshape(_TOKENS)
    return (loss11[0, 0], zq.reshape(z.shape), perp11[0, 0],
            idx_flat.reshape(z.shape[:-1]))


# speedup vs baseline: 1.0640x; 1.0640x over previous
import jax
import jax.numpy as jnp
from jax import lax
from jax.experimental import pallas as pl
from jax.experimental.pallas import tpu as pltpu
from jax.experimental.pallas import tpu_sc as plsc

_N_E = 512
_E_DIM = 256
_BETA = 0.25
_TOKENS = 16384
_TB = 4096
_GRID = _TOKENS // _TB

_NC = 2
_NS = 16
_NW = _NC * _NS
_BPW = _TOKENS // _NW
_CH = 128
_NCH = _BPW // _CH
_NBUF = 3


def _dist_kernel(z_ref, e_ref, idx_ref, counts_ref, msum_ref, loss_ref,
                 perp_ref):
    i = pl.program_id(0)
    z = z_ref[...]
    e = e_ref[...]
    zsq = jnp.sum(z * z, axis=1, keepdims=True)
    esq = jnp.sum(e * e, axis=1)
    mm = lax.dot_general(z, e, (((1,), (1,)), ((), ())),
                         preferred_element_type=jnp.float32)
    d = (zsq + esq[None, :]) - 2.0 * mm
    dmin = jnp.min(d, axis=1, keepdims=True)
    iota_f = lax.broadcasted_iota(jnp.int32, (_TB, _N_E), 1).astype(jnp.float32)
    idxf = jnp.min(jnp.where(d == dmin, iota_f, float(_N_E)),
                   axis=1, keepdims=True)
    idx_ref[...] = idxf.astype(jnp.int32).reshape(_TB // 128, 128)

    cpart = jnp.sum((idxf == iota_f).astype(jnp.float32), axis=0)

    @pl.when(i == 0)
    def _init():
        counts_ref[...] = jnp.zeros_like(counts_ref)
        msum_ref[...] = jnp.zeros_like(msum_ref)

    counts_ref[...] += cpart[None, :]
    msum_ref[...] += jnp.reshape(jnp.sum(dmin), (1, 1))

    @pl.when(i == _GRID - 1)
    def _epilogue():
        g = lax.dot_general(e, e, (((1,), (1,)), ((), ())),
                            preferred_element_type=jnp.float32)
        sq = esq[:, None] + esq[None, :] - 2.0 * g
        ed = jnp.sqrt(jnp.maximum(sq, 0.0))
        ri = lax.broadcasted_iota(jnp.int32, (_N_E, _N_E), 0)
        ci = lax.broadcasted_iota(jnp.int32, (_N_E, _N_E), 1)
        tril = jnp.where(ri >= ci, ed, 0.0)
        e_loss = jnp.exp(-(jnp.sum(tril) / float(_N_E * _N_E)) / 0.1)

        emean = counts_ref[0, :] * (1.0 / float(_TOKENS))
        perp = jnp.exp(-jnp.sum(emean * jnp.log(emean + 1e-10)))

        mse = msum_ref[0, 0] / float(_TOKENS * _E_DIM)
        loss_ref[...] = jnp.reshape((1.0 + _BETA) * mse + e_loss, (1, 1))
        perp_ref[...] = jnp.reshape(perp, (1, 1))


def _gather_body(table_hbm, idx_hbm, out_hbm, idx_v, rows_a, rows_b, rows_c,
                 gsem_a, gsem_b, gsem_c, osem_a, osem_b, osem_c):
    wid = lax.axis_index("s") * _NC + lax.axis_index("c")
    base = wid * _BPW
    pltpu.sync_copy(idx_hbm.at[pl.ds(base, _BPW)], idx_v)
    rows = (rows_a, rows_b, rows_c)
    gsems = (gsem_a, gsem_b, gsem_c)
    osems = (osem_a, osem_b, osem_c)
    gathers = [None] * _NCH
    scatters = [None] * _NCH
    for c in range(min(_NBUF, _NCH)):
        gathers[c] = pltpu.async_copy(
            table_hbm.at[idx_v.at[pl.ds(c * _CH, _CH)]], rows[c % _NBUF],
            gsems[c % _NBUF])
    for c in range(_NCH):
        b = c % _NBUF
        gathers[c].wait()
        scatters[c] = pltpu.async_copy(
            rows[b], out_hbm.at[pl.ds(base + c * _CH, _CH)], osems[b])
        if c + _NBUF < _NCH:
            scatters[c].wait()
            gathers[c + _NBUF] = pltpu.async_copy(
                table_hbm.at[idx_v.at[pl.ds((c + _NBUF) * _CH, _CH)]],
                rows[b], gsems[b])
    for c in range(max(0, _NCH - _NBUF), _NCH):
        scatters[c].wait()


def _sc_gather(table, idx_flat):
    mesh = plsc.VectorSubcoreMesh(core_axis_name="c", subcore_axis_name="s")
    return pl.kernel(
        _gather_body,
        out_type=jax.ShapeDtypeStruct((_TOKENS, _E_DIM), jnp.float32),
        mesh=mesh,
        scratch_types=[
            pltpu.VMEM((_BPW,), jnp.int32),
            pltpu.VMEM((_CH, _E_DIM), jnp.float32),
            pltpu.VMEM((_CH, _E_DIM), jnp.float32),
            pltpu.VMEM((_CH, _E_DIM), jnp.float32),
            pltpu.SemaphoreType.DMA,
            pltpu.SemaphoreType.DMA,
            pltpu.SemaphoreType.DMA,
            pltpu.SemaphoreType.DMA,
            pltpu.SemaphoreType.DMA,
            pltpu.SemaphoreType.DMA,
        ],
    )(table, idx_flat)


def _distance_call(z_flat, e):
    return pl.pallas_call(
        _dist_kernel,
        grid=(_GRID,),
        in_specs=[
            pl.BlockSpec((_TB, _E_DIM), lambda i: (i, 0)),
            pl.BlockSpec((_N_E, _E_DIM), lambda i: (0, 0)),
        ],
        out_specs=[
            pl.BlockSpec((_TB // 128, 128), lambda i: (i, 0)),
            pl.BlockSpec((1, _N_E), lambda i: (0, 0)),
            pl.BlockSpec((1, 1), lambda i: (0, 0)),
            pl.BlockSpec((1, 1), lambda i: (0, 0)),
            pl.BlockSpec((1, 1), lambda i: (0, 0)),
        ],
        out_shape=[
            jax.ShapeDtypeStruct((_TOKENS // 128, 128), jnp.int32),
            jax.ShapeDtypeStruct((1, _N_E), jnp.float32),
            jax.ShapeDtypeStruct((1, 1), jnp.float32),
            jax.ShapeDtypeStruct((1, 1), jnp.float32),
            jax.ShapeDtypeStruct((1, 1), jnp.float32),
        ],
    )(z_flat, e)


def kernel(z, embedding_weight):
    z_flat = z.reshape(_TOKENS, _E_DIM)
    idx2d, _, _, loss11, perp11 = _distance_call(z_flat, embedding_weight)
    idx_flat = idx2d.reshape(_TOKENS)
    zq = _sc_gather(embedding_weight, idx_flat)
    return (loss11[0, 0], zq.reshape(z.shape), perp11[0, 0],
            idx_flat.reshape(z.shape[:-1]))
